# baseline (speedup 1.0000x reference)
import functools

import jax
import jax.numpy as jnp
from jax import lax
from jax.experimental import pallas as pl
from jax.experimental.pallas import tpu as pltpu
from jax.experimental.pallas import tpu_sc as plsc

_BATCH = 64


def _sc_gather(B, D, NC, NS):
    NW = NC * NS
    b_per_w = B // NW
    nb = b_per_w // _BATCH
    mesh = plsc.VectorSubcoreMesh(core_axis_name="c", subcore_axis_name="s")

    @functools.partial(
        pl.kernel,
        out_type=jax.ShapeDtypeStruct((B, D), jnp.float32),
        mesh=mesh,
        compiler_params=pltpu.CompilerParams(use_tc_tiling_on_sc=True),
        scratch_types=[
            pltpu.VMEM((b_per_w,), jnp.int32),
            pltpu.VMEM((b_per_w, D), jnp.float32),
            pltpu.SemaphoreType.DMA,
            pltpu.SemaphoreType.DMA,
        ],
    )
    def gather_kernel(table_hbm, idx_hbm, out_hbm, idx_v, rows_v, sem0, sem1):
        wid = lax.axis_index("s") * NC + lax.axis_index("c")
        base = wid * b_per_w
        pltpu.sync_copy(idx_hbm.at[pl.ds(base, b_per_w)], idx_v)
        sems = (sem0, sem1)

        def wait_batch(s):
            pltpu.make_async_copy(
                table_hbm.at[pl.ds(0, _BATCH // 2)],
                rows_v.at[pl.ds(0, _BATCH // 2)],
                s,
            ).wait()

        for g in range(nb):
            for h in range(_BATCH // 16):
                v16 = idx_v[pl.ds(g * _BATCH + h * 16, 16)]
                for j in range(16):
                    pltpu.async_copy(
                        table_hbm.at[v16[j]],
                        rows_v.at[g * _BATCH + h * 16 + j],
                        sems[j % 2],
                    )
            if g > 0:
                wait_batch(sem0)
                wait_batch(sem1)
        wait_batch(sem0)
        wait_batch(sem1)

        pltpu.sync_copy(rows_v, out_hbm.at[pl.ds(base, b_per_w)])

    return gather_kernel


def kernel(emb, node_index):
    V, D = emb.shape
    (B,) = node_index.shape
    info = plsc.get_sparse_core_info()
    NC, NS = info.num_cores, info.num_subcores
    return _sc_gather(B, D, NC, NS)(emb, node_index.astype(jnp.int32))


# speedup vs baseline: 1.0083x; 1.0083x over previous
import functools

import jax
import jax.numpy as jnp
from jax import lax
from jax.experimental import pallas as pl
from jax.experimental.pallas import tpu as pltpu
from jax.experimental.pallas import tpu_sc as plsc

_BATCH = 64


def _sc_gather(B, D, NC, NS):
    NW = NC * NS
    b_per_w = B // NW
    nb = b_per_w // _BATCH
    mesh = plsc.VectorSubcoreMesh(core_axis_name="c", subcore_axis_name="s")

    @functools.partial(
        pl.kernel,
        out_type=jax.ShapeDtypeStruct((B, D), jnp.float32),
        mesh=mesh,
        compiler_params=pltpu.CompilerParams(use_tc_tiling_on_sc=True),
        scratch_types=[
            pltpu.VMEM((b_per_w,), jnp.int32),
            pltpu.VMEM((b_per_w, D), jnp.float32),
            pltpu.SemaphoreType.DMA,
        ],
    )
    def gather_kernel(table_hbm, idx_hbm, out_hbm, idx_v, rows_v, sem):
        wid = lax.axis_index("s") * NC + lax.axis_index("c")
        base = wid * b_per_w
        pltpu.sync_copy(idx_hbm.at[pl.ds(base, b_per_w)], idx_v)

        def wait_batch():
            pltpu.make_async_copy(
                table_hbm.at[pl.ds(0, _BATCH)],
                rows_v.at[pl.ds(0, _BATCH)],
                sem,
            ).wait()

        for g in range(nb):
            for h in range(_BATCH // 16):
                v16 = idx_v[pl.ds(g * _BATCH + h * 16, 16)]
                for j in range(16):
                    pltpu.async_copy(
                        table_hbm.at[v16[j]],
                        rows_v.at[g * _BATCH + h * 16 + j],
                        sem,
                    )
            if g > 0:
                wait_batch()
        wait_batch()

        pltpu.sync_copy(rows_v, out_hbm.at[pl.ds(base, b_per_w)])

    return gather_kernel


def kernel(emb, node_index):
    V, D = emb.shape
    (B,) = node_index.shape
    info = plsc.get_sparse_core_info()
    NC, NS = info.num_cores, info.num_subcores
    return _sc_gather(B, D, NC, NS)(emb, node_index.astype(jnp.int32))
